# Initial kernel scaffold; baseline (speedup 1.0000x reference)
import jax
import jax.numpy as jnp
from jax.experimental import pallas as pl


def kernel(x):
    raise NotImplementedError("write your pallas kernel here")


# speedup vs baseline: 40.1443x; 40.1443x over previous
import jax
import jax.numpy as jnp
from jax.experimental import pallas as pl
from jax.experimental.pallas import tpu as pltpu

_B, _S, _C = 4, 8192, 768
_K = 8
_CH = 512
_NCH = _S // _CH
_ROWS = _CH + _K

_NEG_INF = float("-inf")


def _topk_body(x_ref, out_ref, comb_v, comb_i):
    j = pl.program_id(1)

    @pl.when(j == 0)
    def _init():
        comb_v[0:_K, :] = jnp.full((_K, _C), _NEG_INF, jnp.float32)
        comb_i[0:_K, :] = jnp.full((_K, _C), -1, jnp.int32)

    comb_v[_K:, :] = x_ref[0]
    row_iota = jax.lax.broadcasted_iota(jnp.int32, (_CH, _C), 0)
    comb_i[_K:, :] = row_iota + j * _CH

    v = comb_v[...]
    idx = comb_i[...]
    new_v = []
    new_i = []
    for r in range(_K):
        m = jnp.max(v, axis=0, keepdims=True)
        eq = v == m
        p = jnp.max(jnp.where(eq, idx, -1), axis=0, keepdims=True)
        new_v.append(m)
        new_i.append(p)
        if r < _K - 1:
            v = jnp.where(idx == p, _NEG_INF, v)

    acc_v = jnp.concatenate(new_v, axis=0)
    acc_i = jnp.concatenate(new_i, axis=0)
    comb_v[0:_K, :] = acc_v
    comb_i[0:_K, :] = acc_i

    @pl.when(j == _NCH - 1)
    def _emit():
        av = acc_v
        ai = acc_i
        outs = []
        for r in range(_K):
            mi = jnp.min(ai, axis=0, keepdims=True)
            val = jnp.max(jnp.where(ai == mi, av, _NEG_INF), axis=0,
                          keepdims=True)
            outs.append(val)
            if r < _K - 1:
                ai = jnp.where(ai == mi, jnp.int32(2**31 - 1), ai)
        out_ref[0] = jnp.concatenate(outs, axis=0)


def kernel(x):
    return pl.pallas_call(
        _topk_body,
        grid=(_B, _NCH),
        in_specs=[pl.BlockSpec((1, _CH, _C), lambda b, j: (b, j, 0))],
        out_specs=pl.BlockSpec((1, _K, _C), lambda b, j: (b, 0, 0)),
        out_shape=jax.ShapeDtypeStruct((_B, _K, _C), jnp.float32),
        scratch_shapes=[
            pltpu.VMEM((_ROWS, _C), jnp.float32),
            pltpu.VMEM((_ROWS, _C), jnp.int32),
        ],
    )(x)


# speedup vs baseline: 42.5375x; 1.0596x over previous
import functools

import jax
import jax.numpy as jnp
from jax import lax
from jax.experimental import pallas as pl
from jax.experimental.pallas import tpu as pltpu
from jax.experimental.pallas import tpu_sc as plsc

_B, _S, _C = 4, 8192, 768
_K = 8
_G = _S // _K
_SG = _G // _K
_NCAND = _K * _K

_CH_A = 2048
_NCH_A = _S // _CH_A
_GPC = _CH_A // _K

_NEG_INF = float("-inf")
_I32_MAX = 2**31 - 1

_NW = 32
_NPW = _B * _NCAND * _C // _NW


def _stage_a_body(x_ref, idx_ref, l1_ref):
    j = pl.program_id(1)
    b = pl.program_id(0)

    x3 = x_ref[0].reshape(_GPC, _K, _C)
    l1_ref[pl.ds(j * _GPC, _GPC), :] = jnp.max(x3, axis=1)

    @pl.when(j == _NCH_A - 1)
    def _select():
        l1v = l1_ref[...].reshape(_SG, _K, _C)
        l2 = jnp.max(l1v, axis=1)

        sg_iota = lax.broadcasted_iota(jnp.int32, (_SG, _C), 0)
        sels = []
        for r in range(_K):
            m = jnp.max(l2, axis=0, keepdims=True)
            p = jnp.max(jnp.where(l2 == m, sg_iota, -1), axis=0,
                        keepdims=True)
            sels.append(p)
            if r < _K - 1:
                l2 = jnp.where(sg_iota == p, _NEG_INF, l2)

        sg_iota3 = lax.broadcasted_iota(jnp.int32, (_SG, 1, _C), 0)
        row8 = lax.broadcasted_iota(jnp.int32, (_K, _C), 0)
        cand_v = []
        cand_g = []
        for r in range(_K):
            eq = sg_iota3 == sels[r].reshape(1, 1, _C)
            cand_v.append(jnp.max(jnp.where(eq, l1v, _NEG_INF), axis=0))
            cand_g.append(sels[r] * _K + row8)
        cv = jnp.concatenate(cand_v, axis=0)
        cg = jnp.concatenate(cand_g, axis=0)

        lane = lax.broadcasted_iota(jnp.int32, (_K, _C), 1)
        outs = []
        for r in range(_K):
            m = jnp.max(cv, axis=0, keepdims=True)
            g = jnp.max(jnp.where(cv == m, cg, -1), axis=0, keepdims=True)
            if r < _K - 1:
                cv = jnp.where(cg == g, _NEG_INF, cv)
            s = g * _K + row8
            outs.append((b * _S + s) * _C + lane)
        idx_ref[0] = jnp.concatenate(outs, axis=0)


def _stage_a(x):
    return pl.pallas_call(
        _stage_a_body,
        grid=(_B, _NCH_A),
        in_specs=[pl.BlockSpec((1, _CH_A, _C), lambda b, j: (b, j, 0))],
        out_specs=pl.BlockSpec((1, _NCAND, _C), lambda b, j: (b, 0, 0)),
        out_shape=jax.ShapeDtypeStruct((_B, _NCAND, _C), jnp.int32),
        scratch_shapes=[pltpu.VMEM((_G, _C), jnp.float32)],
    )(x)


def _stage_b_body(xf_hbm, idxf_hbm, out_hbm, idx_v, val_v, sem):
    wid = lax.axis_index("s") * 2 + lax.axis_index("c")
    base = wid * _NPW
    pltpu.sync_copy(idxf_hbm.at[pl.ds(base, _NPW)], idx_v)
    pltpu.async_copy(xf_hbm.at[idx_v], val_v, sem).wait()
    pltpu.sync_copy(val_v, out_hbm.at[pl.ds(base, _NPW)])


def _stage_b(xf, idxf):
    mesh = plsc.VectorSubcoreMesh(core_axis_name="c", subcore_axis_name="s")
    run = functools.partial(
        pl.kernel,
        mesh=mesh,
        out_type=jax.ShapeDtypeStruct((_B * _NCAND * _C,), jnp.float32),
        scratch_types=[
            pltpu.VMEM((_NPW,), jnp.int32),
            pltpu.VMEM((_NPW,), jnp.float32),
            pltpu.SemaphoreType.DMA,
        ],
    )(_stage_b_body)
    return run(xf, idxf)


def _stage_c_body(g_ref, idx_ref, out_ref):
    b = pl.program_id(0)
    gv = g_ref[0]
    s = idx_ref[0] // _C - b * _S

    kept_v = []
    kept_s = []
    for r in range(_K):
        m = jnp.max(gv, axis=0, keepdims=True)
        p = jnp.max(jnp.where(gv == m, s, -1), axis=0, keepdims=True)
        kept_v.append(m)
        kept_s.append(p)
        if r < _K - 1:
            gv = jnp.where(s == p, _NEG_INF, gv)
    av = jnp.concatenate(kept_v, axis=0)
    ai = jnp.concatenate(kept_s, axis=0)

    outs = []
    for r in range(_K):
        mi = jnp.min(ai, axis=0, keepdims=True)
        outs.append(jnp.max(jnp.where(ai == mi, av, _NEG_INF), axis=0,
                            keepdims=True))
        if r < _K - 1:
            ai = jnp.where(ai == mi, _I32_MAX, ai)
    out_ref[0] = jnp.concatenate(outs, axis=0)


def _stage_c(gathered, idx):
    return pl.pallas_call(
        _stage_c_body,
        grid=(_B,),
        in_specs=[
            pl.BlockSpec((1, _NCAND, _C), lambda b: (b, 0, 0)),
            pl.BlockSpec((1, _NCAND, _C), lambda b: (b, 0, 0)),
        ],
        out_specs=pl.BlockSpec((1, _K, _C), lambda b: (b, 0, 0)),
        out_shape=jax.ShapeDtypeStruct((_B, _K, _C), jnp.float32),
    )(gathered, idx)


def kernel(x):
    idx = _stage_a(x)
    gathered = _stage_b(x.reshape(-1), idx.reshape(-1))
    return _stage_c(gathered.reshape(_B, _NCAND, _C), idx)
